# Initial kernel scaffold; baseline (speedup 1.0000x reference)
import jax
import jax.numpy as jnp
from jax.experimental import pallas as pl


def kernel(x_a, x_b, edge_index_ab, edge_index_ba, W_enc_a, b_enc_a, W_enc_b, b_enc_b, W_self_a1, W_nbr_a1, W_self_b1, W_nbr_b1, W_self_a2, W_nbr_a2, W_self_b2, W_nbr_b2, W_head, b_head):
    raise NotImplementedError("write your pallas kernel here")


# speedup vs baseline: 4.7997x; 4.7997x over previous
import functools

import jax
import jax.numpy as jnp
from jax import lax
from jax.experimental import pallas as pl
from jax.experimental.pallas import tpu as pltpu
from jax.experimental.pallas import tpu_sc as plsc

N = 10000
E = 160000
C = 128
K = 80
NSUB = 16
NCORE = 2
RPT = 632
RPT_LAST = N - 15 * RPT

f32 = jnp.float32
i32 = jnp.int32


def _per_tile_rows(sub, fn):
    r0 = sub * RPT

    @pl.when(sub < NSUB - 1)
    def _():
        fn(r0, RPT)

    @pl.when(sub == NSUB - 1)
    def _():
        fn(r0, RPT_LAST)


def _sc_layer1(h_a, h_b, s_ab, d_ab, s_ba, d_ba, z128):
    mesh = plsc.VectorSubcoreMesh(core_axis_name="c", subcore_axis_name="s")

    @functools.partial(
        pl.kernel,
        out_type=[
            jax.ShapeDtypeStruct((N, C), f32),
            jax.ShapeDtypeStruct((N,), f32),
            jax.ShapeDtypeStruct((N, C), f32),
            jax.ShapeDtypeStruct((N,), f32),
        ],
        mesh=mesh,
        scratch_types=[
            pltpu.VMEM_SHARED((N, C), f32),
            pltpu.VMEM_SHARED((N,), f32),
            pltpu.VMEM((K,), i32),
            pltpu.VMEM((K,), i32),
            pltpu.VMEM((K, C), f32),
            pltpu.VMEM((K,), f32),
            pltpu.VMEM((640,), f32),
            pltpu.SemaphoreType.DMA,
        ],
    )
    def k(h_a_hbm, h_b_hbm, sab_hbm, dab_hbm, sba_hbm, dba_hbm,
          z128_hbm,
          sum_b_out, cnt_b_out, sum_a_out, cnt_a_out,
          acc, cnt, idx_s, idx_d, rows, ones_v, stage, sem):
        core = lax.axis_index("c")
        sub = lax.axis_index("s")
        _per_tile_rows(sub, lambda r0, nr: pltpu.sync_copy(
            z128_hbm.at[pl.ds(r0, nr)], acc.at[pl.ds(r0, nr)]))
        zv = jnp.zeros((16,), f32)
        for g in range(0, 640, 16):
            stage[pl.ds(g, 16)] = zv
        for g in range(0, K, 16):
            ones_v[pl.ds(g, 16)] = jnp.ones((16,), f32)
        _per_tile_rows(sub, lambda r0, nr: pltpu.sync_copy(
            stage.at[pl.ds(0, nr)], cnt.at[pl.ds(r0, nr)]))
        plsc.subcore_barrier()

        def run(h_hbm, s_hbm, d_hbm):
            base = sub * (E // NSUB)

            def body(i, carry):
                off = base + i * K
                pltpu.sync_copy(s_hbm.at[pl.ds(off, K)], idx_s)
                pltpu.sync_copy(d_hbm.at[pl.ds(off, K)], idx_d)
                pltpu.async_copy(h_hbm.at[idx_s], rows, sem).wait()
                pltpu.sync_copy(rows, acc.at[idx_d], add=True)
                pltpu.sync_copy(ones_v, cnt.at[idx_d], add=True)
                return carry

            lax.fori_loop(0, (E // NSUB) // K, body, 0)

        @pl.when(core == 0)
        def _():
            run(h_a_hbm, sab_hbm, dab_hbm)

        @pl.when(core == 1)
        def _():
            run(h_b_hbm, sba_hbm, dba_hbm)

        plsc.subcore_barrier()

        def wb(sum_out, cnt_out):
            _per_tile_rows(sub, lambda r0, nr: pltpu.sync_copy(
                acc.at[pl.ds(r0, nr)], sum_out.at[pl.ds(r0, nr)]))

            def cnt_wb(r0, nr):
                pltpu.sync_copy(cnt.at[pl.ds(r0, nr)], stage.at[pl.ds(0, nr)])
                pltpu.sync_copy(stage.at[pl.ds(0, nr)], cnt_out.at[pl.ds(r0, nr)])

            _per_tile_rows(sub, cnt_wb)

        @pl.when(core == 0)
        def _():
            wb(sum_b_out, cnt_b_out)

        @pl.when(core == 1)
        def _():
            wb(sum_a_out, cnt_a_out)

    return k(h_a, h_b, s_ab, d_ab, s_ba, d_ba, z128)


_EPT = E // (NCORE * NSUB)
_NCH2 = _EPT // K
_KT = _EPT - _NCH2 * K


def _sc_layer2(h_b1, s_ba, d_ba, z128):
    mesh = plsc.VectorSubcoreMesh(core_axis_name="c", subcore_axis_name="s")

    @functools.partial(
        pl.kernel,
        out_type=jax.ShapeDtypeStruct((NCORE, N, C), f32),
        mesh=mesh,
        scratch_types=[
            pltpu.VMEM_SHARED((N, C), f32),
            pltpu.VMEM((K,), i32),
            pltpu.VMEM((K,), i32),
            pltpu.VMEM((K, C), f32),
            pltpu.VMEM((_KT,), i32),
            pltpu.VMEM((_KT,), i32),
            pltpu.VMEM((_KT, C), f32),
            pltpu.SemaphoreType.DMA,
        ],
    )
    def k(h_hbm, s_hbm, d_hbm, z128_hbm, part_out,
          acc, idx_s, idx_d, rows, idx_st, idx_dt, rows_t, sem):
        core = lax.axis_index("c")
        sub = lax.axis_index("s")
        _per_tile_rows(sub, lambda r0, nr: pltpu.sync_copy(
            z128_hbm.at[pl.ds(r0, nr)], acc.at[pl.ds(r0, nr)]))
        plsc.subcore_barrier()

        base = (core * NSUB + sub) * _EPT

        def body(i, carry):
            off = base + i * K
            pltpu.sync_copy(s_hbm.at[pl.ds(off, K)], idx_s)
            pltpu.sync_copy(d_hbm.at[pl.ds(off, K)], idx_d)
            pltpu.async_copy(h_hbm.at[idx_s], rows, sem).wait()
            pltpu.sync_copy(rows, acc.at[idx_d], add=True)
            return carry

        lax.fori_loop(0, _NCH2, body, 0)
        off = base + _NCH2 * K
        pltpu.sync_copy(s_hbm.at[pl.ds(off, _KT)], idx_st)
        pltpu.sync_copy(d_hbm.at[pl.ds(off, _KT)], idx_dt)
        pltpu.async_copy(h_hbm.at[idx_st], rows_t, sem).wait()
        pltpu.sync_copy(rows_t, acc.at[idx_dt], add=True)

        plsc.subcore_barrier()
        _per_tile_rows(sub, lambda r0, nr: pltpu.sync_copy(
            acc.at[pl.ds(r0, nr)], part_out.at[core, pl.ds(r0, nr)]))

    return k(h_b1, s_ba, d_ba, z128)


RB = 1000


def _enc_body(xa, xb, wa, ba, wb, bb, ha, hb):
    ha[...] = jnp.maximum(
        jnp.dot(xa[...], wa[...], preferred_element_type=f32) + ba[...], 0.0)
    hb[...] = jnp.maximum(
        jnp.dot(xb[...], wb[...], preferred_element_type=f32) + bb[...], 0.0)


def _encoder(x_a, x_b, W_a, b_a, W_b, b_b):
    row = pl.BlockSpec((RB, C), lambda i: (i, 0))
    w = pl.BlockSpec((C, C), lambda i: (0, 0))
    b = pl.BlockSpec((1, C), lambda i: (0, 0))
    return pl.pallas_call(
        _enc_body,
        grid=(N // RB,),
        in_specs=[row, row, w, b, w, b],
        out_specs=[row, row],
        out_shape=[jax.ShapeDtypeStruct((N, C), f32)] * 2,
    )(x_a, x_b, W_a, b_a, W_b, b_b)


def _l1_body(ha, hb, sa, ca, sb, cb, wsa, wna, wsb, wnb, ha1, hb1):
    ma = sa[...] / jnp.maximum(ca[...], 1.0)
    mb = sb[...] / jnp.maximum(cb[...], 1.0)
    ha1[...] = jnp.maximum(
        jnp.dot(ha[...], wsa[...], preferred_element_type=f32)
        + jnp.dot(ma, wna[...], preferred_element_type=f32), 0.0)
    hb1[...] = jnp.maximum(
        jnp.dot(hb[...], wsb[...], preferred_element_type=f32)
        + jnp.dot(mb, wnb[...], preferred_element_type=f32), 0.0)


def _combine1(h_a, h_b, s_a, c_a, s_b, c_b, wsa, wna, wsb, wnb):
    row = pl.BlockSpec((RB, C), lambda i: (i, 0))
    cnt = pl.BlockSpec((RB, 1), lambda i: (i, 0))
    w = pl.BlockSpec((C, C), lambda i: (0, 0))
    return pl.pallas_call(
        _l1_body,
        grid=(N // RB,),
        in_specs=[row, row, row, cnt, row, cnt, w, w, w, w],
        out_specs=[row, row],
        out_shape=[jax.ShapeDtypeStruct((N, C), f32)] * 2,
    )(h_a, h_b, s_a, c_a, s_b, c_b, wsa, wna, wsb, wnb)


def _l2_body(ha1, p0, p1, ca, ws, wn, wh, bh, out):
    m = (p0[0] + p1[0]) / jnp.maximum(ca[...], 1.0)
    h2 = jnp.maximum(
        jnp.dot(ha1[...], ws[...], preferred_element_type=f32)
        + jnp.dot(m, wn[...], preferred_element_type=f32), 0.0)
    out[...] = jnp.dot(h2, wh[...], preferred_element_type=f32) + bh[...]


def _final(h_a1, parts, c_a, ws, wn, wh, bh):
    row = pl.BlockSpec((RB, C), lambda i: (i, 0))
    p0 = pl.BlockSpec((1, RB, C), lambda i: (0, i, 0))
    p1 = pl.BlockSpec((1, RB, C), lambda i: (1, i, 0))
    cnt = pl.BlockSpec((RB, 1), lambda i: (i, 0))
    w = pl.BlockSpec((C, C), lambda i: (0, 0))
    whs = pl.BlockSpec((C, 1), lambda i: (0, 0))
    bhs = pl.BlockSpec((1, 1), lambda i: (0, 0))
    return pl.pallas_call(
        _l2_body,
        grid=(N // RB,),
        in_specs=[row, p0, p1, cnt, w, w, whs, bhs],
        out_specs=pl.BlockSpec((RB, 1), lambda i: (i, 0)),
        out_shape=jax.ShapeDtypeStruct((N, 1), f32),
    )(h_a1, parts, parts, c_a, ws, wn, wh, bh)


def kernel(x_a, x_b, edge_index_ab, edge_index_ba,
           W_enc_a, b_enc_a, W_enc_b, b_enc_b,
           W_self_a1, W_nbr_a1, W_self_b1, W_nbr_b1,
           W_self_a2, W_nbr_a2, W_self_b2, W_nbr_b2,
           W_head, b_head):
    z128 = jnp.zeros((N, C), f32)

    h_a, h_b = _encoder(x_a, x_b, W_enc_a, b_enc_a.reshape(1, C),
                        W_enc_b, b_enc_b.reshape(1, C))
    s_ab, d_ab = edge_index_ab[0], edge_index_ab[1]
    s_ba, d_ba = edge_index_ba[0], edge_index_ba[1]
    sum_b, cnt_b, sum_a, cnt_a = _sc_layer1(
        h_a, h_b, s_ab, d_ab, s_ba, d_ba, z128)
    h_a1, h_b1 = _combine1(h_a, h_b, sum_a, cnt_a[:, None], sum_b,
                           cnt_b[:, None],
                           W_self_a1, W_nbr_a1, W_self_b1, W_nbr_b1)
    parts = _sc_layer2(h_b1, s_ba, d_ba, z128)
    return _final(h_a1, parts, cnt_a[:, None], W_self_a2, W_nbr_a2,
                  W_head, b_head.reshape(1, 1))


# speedup vs baseline: 6.9006x; 1.4377x over previous
import functools

import jax
import jax.numpy as jnp
from jax import lax
from jax.experimental import pallas as pl
from jax.experimental.pallas import tpu as pltpu
from jax.experimental.pallas import tpu_sc as plsc

N = 10000
E = 160000
C = 128
K = 80
NSUB = 16
NCORE = 2
RPT = 632
RPT_LAST = N - 15 * RPT

f32 = jnp.float32
i32 = jnp.int32


def _per_tile_rows(sub, fn):
    r0 = sub * RPT

    @pl.when(sub < NSUB - 1)
    def _():
        fn(r0, RPT)

    @pl.when(sub == NSUB - 1)
    def _():
        fn(r0, RPT_LAST)


def _agg_pipeline(h_hbm, s_hbm, d_hbm, base, kk, n, acc,
                  idxA, idxB, rowsA, rowsB,
                  sgA, sgB, ssA, ssB,
                  cnt=None, ones_v=None, scA=None, scB=None):
    def load(idx, c):
        off = base + c * kk
        pltpu.sync_copy(s_hbm.at[pl.ds(off, kk)], idx.at[0])
        pltpu.sync_copy(d_hbm.at[pl.ds(off, kk)], idx.at[1])

    def gather_start(idx, rows, sem):
        pltpu.async_copy(h_hbm.at[idx.at[0]], rows, sem)

    def gather_wait(idx, rows, sem):
        pltpu.make_async_copy(h_hbm.at[idx.at[0]], rows, sem).wait()

    def scatter_start(idx, rows, sem_s, sem_c):
        pltpu.async_copy(rows, acc.at[idx.at[1]], sem_s, add=True)
        if cnt is not None:
            pltpu.async_copy(ones_v, cnt.at[idx.at[1]], sem_c, add=True)

    def scatter_wait(idx, rows, sem_s, sem_c):
        pltpu.make_async_copy(rows, acc.at[idx.at[1]], sem_s).wait()
        if cnt is not None:
            pltpu.make_async_copy(ones_v, cnt.at[idx.at[1]], sem_c).wait()

    load(idxA, 0)
    gather_start(idxA, rowsA, sgA)

    def body(j, carry):
        @pl.when(j > 0)
        def _():
            scatter_wait(idxB, rowsB, ssB, scB)

        load(idxB, 2 * j + 1)
        gather_start(idxB, rowsB, sgB)
        gather_wait(idxA, rowsA, sgA)
        scatter_start(idxA, rowsA, ssA, scA)
        scatter_wait(idxA, rowsA, ssA, scA)
        load(idxA, 2 * j + 2)
        gather_start(idxA, rowsA, sgA)
        gather_wait(idxB, rowsB, sgB)
        scatter_start(idxB, rowsB, ssB, scB)
        return carry

    lax.fori_loop(0, (n - 1) // 2, body, 0)

    scatter_wait(idxB, rowsB, ssB, scB)
    gather_wait(idxA, rowsA, sgA)
    scatter_start(idxA, rowsA, ssA, scA)
    scatter_wait(idxA, rowsA, ssA, scA)


def _sc_layer1(h_a, h_b, s_ab, d_ab, s_ba, d_ba, z128):
    mesh = plsc.VectorSubcoreMesh(core_axis_name="c", subcore_axis_name="s")

    @functools.partial(
        pl.kernel,
        out_type=[
            jax.ShapeDtypeStruct((N, C), f32),
            jax.ShapeDtypeStruct((N,), f32),
            jax.ShapeDtypeStruct((N, C), f32),
            jax.ShapeDtypeStruct((N,), f32),
        ],
        mesh=mesh,
        scratch_types=[
            pltpu.VMEM_SHARED((N, C), f32),
            pltpu.VMEM_SHARED((N,), f32),
            pltpu.VMEM((2, K), i32),
            pltpu.VMEM((2, K), i32),
            pltpu.VMEM((K, C), f32),
            pltpu.VMEM((K, C), f32),
            pltpu.VMEM((K,), f32),
            pltpu.VMEM((640,), f32),
            pltpu.SemaphoreType.DMA,
            pltpu.SemaphoreType.DMA,
            pltpu.SemaphoreType.DMA,
            pltpu.SemaphoreType.DMA,
            pltpu.SemaphoreType.DMA,
            pltpu.SemaphoreType.DMA,
        ],
    )
    def k(h_a_hbm, h_b_hbm, sab_hbm, dab_hbm, sba_hbm, dba_hbm,
          z128_hbm,
          sum_b_out, cnt_b_out, sum_a_out, cnt_a_out,
          acc, cnt, idxA, idxB, rowsA, rowsB, ones_v, stage,
          sgA, sgB, ssA, ssB, scA, scB):
        core = lax.axis_index("c")
        sub = lax.axis_index("s")
        _per_tile_rows(sub, lambda r0, nr: pltpu.sync_copy(
            z128_hbm.at[pl.ds(r0, nr)], acc.at[pl.ds(r0, nr)]))
        zv = jnp.zeros((16,), f32)
        for g in range(0, 640, 16):
            stage[pl.ds(g, 16)] = zv
        for g in range(0, K, 16):
            ones_v[pl.ds(g, 16)] = jnp.ones((16,), f32)
        _per_tile_rows(sub, lambda r0, nr: pltpu.sync_copy(
            stage.at[pl.ds(0, nr)], cnt.at[pl.ds(r0, nr)]))
        plsc.subcore_barrier()

        base = sub * (E // NSUB)
        n = (E // NSUB) // K

        @pl.when(core == 0)
        def _():
            _agg_pipeline(h_a_hbm, sab_hbm, dab_hbm, base, K, n, acc,
                          idxA, idxB, rowsA, rowsB, sgA, sgB, ssA, ssB,
                          cnt=cnt, ones_v=ones_v, scA=scA, scB=scB)

        @pl.when(core == 1)
        def _():
            _agg_pipeline(h_b_hbm, sba_hbm, dba_hbm, base, K, n, acc,
                          idxA, idxB, rowsA, rowsB, sgA, sgB, ssA, ssB,
                          cnt=cnt, ones_v=ones_v, scA=scA, scB=scB)

        plsc.subcore_barrier()

        def wb(sum_out, cnt_out):
            _per_tile_rows(sub, lambda r0, nr: pltpu.sync_copy(
                acc.at[pl.ds(r0, nr)], sum_out.at[pl.ds(r0, nr)]))

            def cnt_wb(r0, nr):
                pltpu.sync_copy(cnt.at[pl.ds(r0, nr)], stage.at[pl.ds(0, nr)])
                pltpu.sync_copy(stage.at[pl.ds(0, nr)], cnt_out.at[pl.ds(r0, nr)])

            _per_tile_rows(sub, cnt_wb)

        @pl.when(core == 0)
        def _():
            wb(sum_b_out, cnt_b_out)

        @pl.when(core == 1)
        def _():
            wb(sum_a_out, cnt_a_out)

    return k(h_a, h_b, s_ab, d_ab, s_ba, d_ba, z128)


_EPT = E // (NCORE * NSUB)
K2 = 40


def _sc_layer2(h_b1, s_ba, d_ba, z128):
    mesh = plsc.VectorSubcoreMesh(core_axis_name="c", subcore_axis_name="s")

    @functools.partial(
        pl.kernel,
        out_type=jax.ShapeDtypeStruct((NCORE, N, C), f32),
        mesh=mesh,
        scratch_types=[
            pltpu.VMEM_SHARED((N, C), f32),
            pltpu.VMEM((2, K2), i32),
            pltpu.VMEM((2, K2), i32),
            pltpu.VMEM((K2, C), f32),
            pltpu.VMEM((K2, C), f32),
            pltpu.SemaphoreType.DMA,
            pltpu.SemaphoreType.DMA,
            pltpu.SemaphoreType.DMA,
            pltpu.SemaphoreType.DMA,
        ],
    )
    def k(h_hbm, s_hbm, d_hbm, z128_hbm, part_out,
          acc, idxA, idxB, rowsA, rowsB, sgA, sgB, ssA, ssB):
        core = lax.axis_index("c")
        sub = lax.axis_index("s")
        _per_tile_rows(sub, lambda r0, nr: pltpu.sync_copy(
            z128_hbm.at[pl.ds(r0, nr)], acc.at[pl.ds(r0, nr)]))
        plsc.subcore_barrier()

        base = (core * NSUB + sub) * _EPT
        _agg_pipeline(h_hbm, s_hbm, d_hbm, base, K2, _EPT // K2, acc,
                      idxA, idxB, rowsA, rowsB, sgA, sgB, ssA, ssB)

        plsc.subcore_barrier()
        _per_tile_rows(sub, lambda r0, nr: pltpu.sync_copy(
            acc.at[pl.ds(r0, nr)], part_out.at[core, pl.ds(r0, nr)]))

    return k(h_b1, s_ba, d_ba, z128)


RB = 1000


def _enc_body(xa, xb, wa, ba, wb, bb, ha, hb):
    ha[...] = jnp.maximum(
        jnp.dot(xa[...], wa[...], preferred_element_type=f32) + ba[...], 0.0)
    hb[...] = jnp.maximum(
        jnp.dot(xb[...], wb[...], preferred_element_type=f32) + bb[...], 0.0)


def _encoder(x_a, x_b, W_a, b_a, W_b, b_b):
    row = pl.BlockSpec((RB, C), lambda i: (i, 0))
    w = pl.BlockSpec((C, C), lambda i: (0, 0))
    b = pl.BlockSpec((1, C), lambda i: (0, 0))
    return pl.pallas_call(
        _enc_body,
        grid=(N // RB,),
        in_specs=[row, row, w, b, w, b],
        out_specs=[row, row],
        out_shape=[jax.ShapeDtypeStruct((N, C), f32)] * 2,
    )(x_a, x_b, W_a, b_a, W_b, b_b)


def _l1_body(ha, hb, sa, ca, sb, cb, wsa, wna, wsb, wnb, ha1, hb1):
    ma = sa[...] / jnp.maximum(ca[...], 1.0)
    mb = sb[...] / jnp.maximum(cb[...], 1.0)
    ha1[...] = jnp.maximum(
        jnp.dot(ha[...], wsa[...], preferred_element_type=f32)
        + jnp.dot(ma, wna[...], preferred_element_type=f32), 0.0)
    hb1[...] = jnp.maximum(
        jnp.dot(hb[...], wsb[...], preferred_element_type=f32)
        + jnp.dot(mb, wnb[...], preferred_element_type=f32), 0.0)


def _combine1(h_a, h_b, s_a, c_a, s_b, c_b, wsa, wna, wsb, wnb):
    row = pl.BlockSpec((RB, C), lambda i: (i, 0))
    cnt = pl.BlockSpec((RB, 1), lambda i: (i, 0))
    w = pl.BlockSpec((C, C), lambda i: (0, 0))
    return pl.pallas_call(
        _l1_body,
        grid=(N // RB,),
        in_specs=[row, row, row, cnt, row, cnt, w, w, w, w],
        out_specs=[row, row],
        out_shape=[jax.ShapeDtypeStruct((N, C), f32)] * 2,
    )(h_a, h_b, s_a, c_a, s_b, c_b, wsa, wna, wsb, wnb)


def _l2_body(ha1, p0, p1, ca, ws, wn, wh, bh, out):
    m = (p0[0] + p1[0]) / jnp.maximum(ca[...], 1.0)
    h2 = jnp.maximum(
        jnp.dot(ha1[...], ws[...], preferred_element_type=f32)
        + jnp.dot(m, wn[...], preferred_element_type=f32), 0.0)
    out[...] = jnp.dot(h2, wh[...], preferred_element_type=f32) + bh[...]


def _final(h_a1, parts, c_a, ws, wn, wh, bh):
    row = pl.BlockSpec((RB, C), lambda i: (i, 0))
    p0 = pl.BlockSpec((1, RB, C), lambda i: (0, i, 0))
    p1 = pl.BlockSpec((1, RB, C), lambda i: (1, i, 0))
    cnt = pl.BlockSpec((RB, 1), lambda i: (i, 0))
    w = pl.BlockSpec((C, C), lambda i: (0, 0))
    whs = pl.BlockSpec((C, 1), lambda i: (0, 0))
    bhs = pl.BlockSpec((1, 1), lambda i: (0, 0))
    return pl.pallas_call(
        _l2_body,
        grid=(N // RB,),
        in_specs=[row, p0, p1, cnt, w, w, whs, bhs],
        out_specs=pl.BlockSpec((RB, 1), lambda i: (i, 0)),
        out_shape=jax.ShapeDtypeStruct((N, 1), f32),
    )(h_a1, parts, parts, c_a, ws, wn, wh, bh)


def kernel(x_a, x_b, edge_index_ab, edge_index_ba,
           W_enc_a, b_enc_a, W_enc_b, b_enc_b,
           W_self_a1, W_nbr_a1, W_self_b1, W_nbr_b1,
           W_self_a2, W_nbr_a2, W_self_b2, W_nbr_b2,
           W_head, b_head):
    z128 = jnp.zeros((N, C), f32)

    h_a, h_b = _encoder(x_a, x_b, W_enc_a, b_enc_a.reshape(1, C),
                        W_enc_b, b_enc_b.reshape(1, C))
    s_ab, d_ab = edge_index_ab[0], edge_index_ab[1]
    s_ba, d_ba = edge_index_ba[0], edge_index_ba[1]
    sum_b, cnt_b, sum_a, cnt_a = _sc_layer1(
        h_a, h_b, s_ab, d_ab, s_ba, d_ba, z128)
    h_a1, h_b1 = _combine1(h_a, h_b, sum_a, cnt_a[:, None], sum_b,
                           cnt_b[:, None],
                           W_self_a1, W_nbr_a1, W_self_b1, W_nbr_b1)
    parts = _sc_layer2(h_b1, s_ba, d_ba, z128)
    return _final(h_a1, parts, cnt_a[:, None], W_self_a2, W_nbr_a2,
                  W_head, b_head.reshape(1, 1))


# speedup vs baseline: 10.1447x; 1.4701x over previous
import functools

import jax
import jax.numpy as jnp
from jax import lax
from jax.experimental import pallas as pl
from jax.experimental.pallas import tpu as pltpu
from jax.experimental.pallas import tpu_sc as plsc

N = 10000
E = 160000
C = 128
K = 128
NCHUNK = E // K
NSUB = 16
NCORE = 2
RPT = 632
RPT_LAST = N - 15 * RPT

f32 = jnp.float32
i32 = jnp.int32


def _per_tile_rows(sub, fn):
    r0 = sub * RPT

    @pl.when(sub < NSUB - 1)
    def _():
        fn(r0, RPT)

    @pl.when(sub == NSUB - 1)
    def _():
        fn(r0, RPT_LAST)


def _agg_pipeline(h_hbm, e_hbm, c0, n, np_pairs, acc,
                  idxA, idxB, rowsA, rowsB,
                  sgA, sgB, ssA, ssB,
                  cnt=None, ones_v=None, scA=None, scB=None):
    def load(idx, c):
        pltpu.sync_copy(e_hbm.at[c0 + c], idx)

    def gather_start(idx, rows, sem):
        pltpu.async_copy(h_hbm.at[idx.at[0]], rows, sem)

    def gather_wait(idx, rows, sem):
        pltpu.make_async_copy(h_hbm.at[idx.at[0]], rows, sem).wait()

    def scatter_start(idx, rows, sem_s, sem_c):
        pltpu.async_copy(rows, acc.at[idx.at[1]], sem_s, add=True)
        if cnt is not None:
            pltpu.async_copy(ones_v, cnt.at[idx.at[1]], sem_c, add=True)

    def scatter_wait(idx, rows, sem_s, sem_c):
        pltpu.make_async_copy(rows, acc.at[idx.at[1]], sem_s).wait()
        if cnt is not None:
            pltpu.make_async_copy(ones_v, cnt.at[idx.at[1]], sem_c).wait()

    load(idxA, 0)
    gather_start(idxA, rowsA, sgA)

    def body(j, carry):
        @pl.when(j > 0)
        def _():
            scatter_wait(idxB, rowsB, ssB, scB)

        load(idxB, 2 * j + 1)
        gather_start(idxB, rowsB, sgB)
        gather_wait(idxA, rowsA, sgA)
        scatter_start(idxA, rowsA, ssA, scA)
        scatter_wait(idxA, rowsA, ssA, scA)

        @pl.when(2 * j + 2 < n)
        def _():
            load(idxA, 2 * j + 2)
            gather_start(idxA, rowsA, sgA)

        gather_wait(idxB, rowsB, sgB)
        scatter_start(idxB, rowsB, ssB, scB)
        return carry

    lax.fori_loop(0, np_pairs, body, 0)

    scatter_wait(idxB, rowsB, ssB, scB)

    @pl.when(2 * np_pairs < n)
    def _():
        gather_wait(idxA, rowsA, sgA)
        scatter_start(idxA, rowsA, ssA, scA)
        scatter_wait(idxA, rowsA, ssA, scA)

    @pl.when(2 * np_pairs + 1 < n)
    def _():
        load(idxB, 2 * np_pairs + 1)
        gather_start(idxB, rowsB, sgB)
        gather_wait(idxB, rowsB, sgB)
        scatter_start(idxB, rowsB, ssB, scB)
        scatter_wait(idxB, rowsB, ssB, scB)


def _sc_layer1(h_a, h_b, e_ab, e_ba, z128):
    mesh = plsc.VectorSubcoreMesh(core_axis_name="c", subcore_axis_name="s")

    @functools.partial(
        pl.kernel,
        out_type=[
            jax.ShapeDtypeStruct((N, C), f32),
            jax.ShapeDtypeStruct((N,), f32),
            jax.ShapeDtypeStruct((N, C), f32),
            jax.ShapeDtypeStruct((N,), f32),
        ],
        mesh=mesh,
        scratch_types=[
            pltpu.VMEM_SHARED((N, C), f32),
            pltpu.VMEM_SHARED((N,), f32),
            pltpu.VMEM((2, K), i32),
            pltpu.VMEM((2, K), i32),
            pltpu.VMEM((K, C), f32),
            pltpu.VMEM((K, C), f32),
            pltpu.VMEM((K,), f32),
            pltpu.VMEM((640,), f32),
            pltpu.SemaphoreType.DMA,
            pltpu.SemaphoreType.DMA,
            pltpu.SemaphoreType.DMA,
            pltpu.SemaphoreType.DMA,
            pltpu.SemaphoreType.DMA,
            pltpu.SemaphoreType.DMA,
        ],
    )
    def k(h_a_hbm, h_b_hbm, eab_hbm, eba_hbm, z128_hbm,
          sum_b_out, cnt_b_out, sum_a_out, cnt_a_out,
          acc, cnt, idxA, idxB, rowsA, rowsB, ones_v, stage,
          sgA, sgB, ssA, ssB, scA, scB):
        core = lax.axis_index("c")
        sub = lax.axis_index("s")
        _per_tile_rows(sub, lambda r0, nr: pltpu.sync_copy(
            z128_hbm.at[pl.ds(r0, nr)], acc.at[pl.ds(r0, nr)]))
        zv = jnp.zeros((16,), f32)
        for g in range(0, 640, 16):
            stage[pl.ds(g, 16)] = zv
        for g in range(0, K, 16):
            ones_v[pl.ds(g, 16)] = jnp.ones((16,), f32)
        _per_tile_rows(sub, lambda r0, nr: pltpu.sync_copy(
            stage.at[pl.ds(0, nr)], cnt.at[pl.ds(r0, nr)]))
        plsc.subcore_barrier()

        c0 = (sub * NCHUNK) // NSUB
        n = ((sub + 1) * NCHUNK) // NSUB - c0
        np_pairs = (NCHUNK // NSUB) // 2

        @pl.when(core == 0)
        def _():
            _agg_pipeline(h_a_hbm, eab_hbm, c0, n, np_pairs, acc,
                          idxA, idxB, rowsA, rowsB, sgA, sgB, ssA, ssB,
                          cnt=cnt, ones_v=ones_v, scA=scA, scB=scB)

        @pl.when(core == 1)
        def _():
            _agg_pipeline(h_b_hbm, eba_hbm, c0, n, np_pairs, acc,
                          idxA, idxB, rowsA, rowsB, sgA, sgB, ssA, ssB,
                          cnt=cnt, ones_v=ones_v, scA=scA, scB=scB)

        plsc.subcore_barrier()

        def wb(sum_out, cnt_out):
            _per_tile_rows(sub, lambda r0, nr: pltpu.sync_copy(
                acc.at[pl.ds(r0, nr)], sum_out.at[pl.ds(r0, nr)]))

            def cnt_wb(r0, nr):
                pltpu.sync_copy(cnt.at[pl.ds(r0, nr)], stage.at[pl.ds(0, nr)])
                pltpu.sync_copy(stage.at[pl.ds(0, nr)], cnt_out.at[pl.ds(r0, nr)])

            _per_tile_rows(sub, cnt_wb)

        @pl.when(core == 0)
        def _():
            wb(sum_b_out, cnt_b_out)

        @pl.when(core == 1)
        def _():
            wb(sum_a_out, cnt_a_out)

    return k(h_a, h_b, e_ab, e_ba, z128)


def _sc_layer2(h_b1, e_ba, z128):
    mesh = plsc.VectorSubcoreMesh(core_axis_name="c", subcore_axis_name="s")

    @functools.partial(
        pl.kernel,
        out_type=jax.ShapeDtypeStruct((NCORE, N, C), f32),
        mesh=mesh,
        scratch_types=[
            pltpu.VMEM_SHARED((N, C), f32),
            pltpu.VMEM((2, K), i32),
            pltpu.VMEM((2, K), i32),
            pltpu.VMEM((K, C), f32),
            pltpu.VMEM((K, C), f32),
            pltpu.SemaphoreType.DMA,
            pltpu.SemaphoreType.DMA,
            pltpu.SemaphoreType.DMA,
            pltpu.SemaphoreType.DMA,
        ],
    )
    def k(h_hbm, eba_hbm, z128_hbm, part_out,
          acc, idxA, idxB, rowsA, rowsB, sgA, sgB, ssA, ssB):
        core = lax.axis_index("c")
        sub = lax.axis_index("s")
        _per_tile_rows(sub, lambda r0, nr: pltpu.sync_copy(
            z128_hbm.at[pl.ds(r0, nr)], acc.at[pl.ds(r0, nr)]))
        plsc.subcore_barrier()

        w = core * NSUB + sub
        c0 = (w * NCHUNK) // (NCORE * NSUB)
        n = ((w + 1) * NCHUNK) // (NCORE * NSUB) - c0
        np_pairs = (NCHUNK // (NCORE * NSUB)) // 2

        _agg_pipeline(h_hbm, eba_hbm, c0, n, np_pairs, acc,
                      idxA, idxB, rowsA, rowsB, sgA, sgB, ssA, ssB)

        plsc.subcore_barrier()
        _per_tile_rows(sub, lambda r0, nr: pltpu.sync_copy(
            acc.at[pl.ds(r0, nr)], part_out.at[core, pl.ds(r0, nr)]))

    return k(h_b1, e_ba, z128)


RB = 1000


def _enc_body(xa, xb, wa, ba, wb, bb, ha, hb):
    ha[...] = jnp.maximum(
        jnp.dot(xa[...], wa[...], preferred_element_type=f32) + ba[...], 0.0)
    hb[...] = jnp.maximum(
        jnp.dot(xb[...], wb[...], preferred_element_type=f32) + bb[...], 0.0)


def _encoder(x_a, x_b, W_a, b_a, W_b, b_b):
    row = pl.BlockSpec((RB, C), lambda i: (i, 0))
    w = pl.BlockSpec((C, C), lambda i: (0, 0))
    b = pl.BlockSpec((1, C), lambda i: (0, 0))
    return pl.pallas_call(
        _enc_body,
        grid=(N // RB,),
        in_specs=[row, row, w, b, w, b],
        out_specs=[row, row],
        out_shape=[jax.ShapeDtypeStruct((N, C), f32)] * 2,
    )(x_a, x_b, W_a, b_a, W_b, b_b)


def _l1_body(ha, hb, sa, ca, sb, cb, wsa, wna, wsb, wnb, ha1, hb1):
    ma = sa[...] / jnp.maximum(ca[...], 1.0)
    mb = sb[...] / jnp.maximum(cb[...], 1.0)
    ha1[...] = jnp.maximum(
        jnp.dot(ha[...], wsa[...], preferred_element_type=f32)
        + jnp.dot(ma, wna[...], preferred_element_type=f32), 0.0)
    hb1[...] = jnp.maximum(
        jnp.dot(hb[...], wsb[...], preferred_element_type=f32)
        + jnp.dot(mb, wnb[...], preferred_element_type=f32), 0.0)


def _combine1(h_a, h_b, s_a, c_a, s_b, c_b, wsa, wna, wsb, wnb):
    row = pl.BlockSpec((RB, C), lambda i: (i, 0))
    cnt = pl.BlockSpec((RB, 1), lambda i: (i, 0))
    w = pl.BlockSpec((C, C), lambda i: (0, 0))
    return pl.pallas_call(
        _l1_body,
        grid=(N // RB,),
        in_specs=[row, row, row, cnt, row, cnt, w, w, w, w],
        out_specs=[row, row],
        out_shape=[jax.ShapeDtypeStruct((N, C), f32)] * 2,
    )(h_a, h_b, s_a, c_a, s_b, c_b, wsa, wna, wsb, wnb)


def _l2_body(ha1, p0, p1, ca, ws, wn, wh, bh, out):
    m = (p0[0] + p1[0]) / jnp.maximum(ca[...], 1.0)
    h2 = jnp.maximum(
        jnp.dot(ha1[...], ws[...], preferred_element_type=f32)
        + jnp.dot(m, wn[...], preferred_element_type=f32), 0.0)
    out[...] = jnp.dot(h2, wh[...], preferred_element_type=f32) + bh[...]


def _final(h_a1, parts, c_a, ws, wn, wh, bh):
    row = pl.BlockSpec((RB, C), lambda i: (i, 0))
    p0 = pl.BlockSpec((1, RB, C), lambda i: (0, i, 0))
    p1 = pl.BlockSpec((1, RB, C), lambda i: (1, i, 0))
    cnt = pl.BlockSpec((RB, 1), lambda i: (i, 0))
    w = pl.BlockSpec((C, C), lambda i: (0, 0))
    whs = pl.BlockSpec((C, 1), lambda i: (0, 0))
    bhs = pl.BlockSpec((1, 1), lambda i: (0, 0))
    return pl.pallas_call(
        _l2_body,
        grid=(N // RB,),
        in_specs=[row, p0, p1, cnt, w, w, whs, bhs],
        out_specs=pl.BlockSpec((RB, 1), lambda i: (i, 0)),
        out_shape=jax.ShapeDtypeStruct((N, 1), f32),
    )(h_a1, parts, parts, c_a, ws, wn, wh, bh)


def kernel(x_a, x_b, edge_index_ab, edge_index_ba,
           W_enc_a, b_enc_a, W_enc_b, b_enc_b,
           W_self_a1, W_nbr_a1, W_self_b1, W_nbr_b1,
           W_self_a2, W_nbr_a2, W_self_b2, W_nbr_b2,
           W_head, b_head):
    z128 = jnp.zeros((N, C), f32)
    e_ab = edge_index_ab.reshape(2, NCHUNK, K).transpose(1, 0, 2)
    e_ba = edge_index_ba.reshape(2, NCHUNK, K).transpose(1, 0, 2)

    h_a, h_b = _encoder(x_a, x_b, W_enc_a, b_enc_a.reshape(1, C),
                        W_enc_b, b_enc_b.reshape(1, C))
    sum_b, cnt_b, sum_a, cnt_a = _sc_layer1(h_a, h_b, e_ab, e_ba, z128)
    h_a1, h_b1 = _combine1(h_a, h_b, sum_a, cnt_a[:, None], sum_b,
                           cnt_b[:, None],
                           W_self_a1, W_nbr_a1, W_self_b1, W_nbr_b1)
    parts = _sc_layer2(h_b1, e_ba, z128)
    return _final(h_a1, parts, cnt_a[:, None], W_self_a2, W_nbr_a2,
                  W_head, b_head.reshape(1, 1))


# speedup vs baseline: 10.9629x; 1.0806x over previous
import functools

import jax
import jax.numpy as jnp
from jax import lax
from jax.experimental import pallas as pl
from jax.experimental.pallas import tpu as pltpu
from jax.experimental.pallas import tpu_sc as plsc

N = 10000
E = 160000
C = 128
K = 128
NCHUNK = E // K
NSUB = 16
NCORE = 2
RPT = 632
RPT_LAST = N - 15 * RPT
NPRE = 40

f32 = jnp.float32
i32 = jnp.int32


def _per_tile_rows(sub, fn):
    r0 = sub * RPT

    @pl.when(sub < NSUB - 1)
    def _():
        fn(r0, RPT)

    @pl.when(sub == NSUB - 1)
    def _():
        fn(r0, RPT_LAST)


def _agg_pipeline(h_hbm, e_hbm, c0, n, np_pairs, npre, acc,
                  idx_all, rowsA, rowsB,
                  sgA, sgB, ssA, ssB,
                  cnt=None, ones_v=None, scA=None, scB=None):
    pltpu.sync_copy(e_hbm.at[pl.ds(c0, npre)], idx_all.at[pl.ds(0, npre)])

    def _when(pred, fn):
        if isinstance(pred, bool):
            if pred:
                fn()
        else:
            pl.when(pred)(fn)

    def gather_start(c, rows, sem):
        pltpu.async_copy(h_hbm.at[idx_all.at[c, 0]], rows, sem)

    def gather_wait(c, rows, sem):
        pltpu.make_async_copy(h_hbm.at[idx_all.at[c, 0]], rows, sem).wait()

    def scatter_start(c, rows, sem_s, sem_c):
        pltpu.async_copy(rows, acc.at[idx_all.at[c, 1]], sem_s, add=True)
        if cnt is not None:
            pltpu.async_copy(ones_v, cnt.at[idx_all.at[c, 1]], sem_c, add=True)

    def scatter_wait(c, rows, sem_s, sem_c):
        pltpu.make_async_copy(rows, acc.at[idx_all.at[c, 1]], sem_s).wait()
        if cnt is not None:
            pltpu.make_async_copy(ones_v, cnt.at[idx_all.at[c, 1]], sem_c).wait()

    gather_start(0, rowsA, sgA)

    def body(j, carry):
        @pl.when(j > 0)
        def _():
            scatter_wait(2 * j - 1, rowsB, ssB, scB)

        gather_start(2 * j + 1, rowsB, sgB)
        gather_wait(2 * j, rowsA, sgA)
        scatter_start(2 * j, rowsA, ssA, scA)
        scatter_wait(2 * j, rowsA, ssA, scA)

        @pl.when(2 * j + 2 < n)
        def _():
            gather_start(2 * j + 2, rowsA, sgA)

        gather_wait(2 * j + 1, rowsB, sgB)
        scatter_start(2 * j + 1, rowsB, ssB, scB)
        return carry

    lax.fori_loop(0, np_pairs, body, 0)

    scatter_wait(2 * np_pairs - 1, rowsB, ssB, scB)

    def _tailA():
        gather_wait(2 * np_pairs, rowsA, sgA)
        scatter_start(2 * np_pairs, rowsA, ssA, scA)
        scatter_wait(2 * np_pairs, rowsA, ssA, scA)

    def _tailB():
        gather_start(2 * np_pairs + 1, rowsB, sgB)
        gather_wait(2 * np_pairs + 1, rowsB, sgB)
        scatter_start(2 * np_pairs + 1, rowsB, ssB, scB)
        scatter_wait(2 * np_pairs + 1, rowsB, ssB, scB)

    _when(2 * np_pairs < n, _tailA)
    _when(2 * np_pairs + 1 < n, _tailB)


def _sc_layer1(h_a, h_b, e_ab, e_ba, z128):
    mesh = plsc.VectorSubcoreMesh(core_axis_name="c", subcore_axis_name="s")

    @functools.partial(
        pl.kernel,
        out_type=[
            jax.ShapeDtypeStruct((N, C), f32),
            jax.ShapeDtypeStruct((N,), f32),
            jax.ShapeDtypeStruct((N, C), f32),
            jax.ShapeDtypeStruct((N,), f32),
        ],
        mesh=mesh,
        scratch_types=[
            pltpu.VMEM_SHARED((N, C), f32),
            pltpu.VMEM_SHARED((N,), f32),
            pltpu.VMEM((NPRE, 2, K), i32),
            pltpu.VMEM((K, C), f32),
            pltpu.VMEM((K, C), f32),
            pltpu.VMEM((K,), f32),
            pltpu.VMEM((640,), f32),
            pltpu.SemaphoreType.DMA,
            pltpu.SemaphoreType.DMA,
            pltpu.SemaphoreType.DMA,
            pltpu.SemaphoreType.DMA,
            pltpu.SemaphoreType.DMA,
            pltpu.SemaphoreType.DMA,
        ],
    )
    def k(h_a_hbm, h_b_hbm, eab_hbm, eba_hbm, z128_hbm,
          sum_b_out, cnt_b_out, sum_a_out, cnt_a_out,
          acc, cnt, idx_all, rowsA, rowsB, ones_v, stage,
          sgA, sgB, ssA, ssB, scA, scB):
        core = lax.axis_index("c")
        sub = lax.axis_index("s")
        _per_tile_rows(sub, lambda r0, nr: pltpu.sync_copy(
            z128_hbm.at[pl.ds(r0, nr)], acc.at[pl.ds(r0, nr)]))
        zv = jnp.zeros((16,), f32)
        for g in range(0, 640, 16):
            stage[pl.ds(g, 16)] = zv
        for g in range(0, K, 16):
            ones_v[pl.ds(g, 16)] = jnp.ones((16,), f32)
        _per_tile_rows(sub, lambda r0, nr: pltpu.sync_copy(
            stage.at[pl.ds(0, nr)], cnt.at[pl.ds(r0, nr)]))
        plsc.subcore_barrier()

        c0 = (sub * NCHUNK) // NSUB
        n = ((sub + 1) * NCHUNK) // NSUB - c0

        def run(h_hbm, e_hbm):
            _agg_pipeline(h_hbm, e_hbm, c0, NPRE, NPRE // 2, NPRE, acc,
                          idx_all, rowsA, rowsB, sgA, sgB, ssA, ssB,
                          cnt=cnt, ones_v=ones_v, scA=scA, scB=scB)
            _agg_pipeline(h_hbm, e_hbm, c0 + NPRE, n - NPRE, 19, NPRE - 1,
                          acc, idx_all, rowsA, rowsB, sgA, sgB, ssA, ssB,
                          cnt=cnt, ones_v=ones_v, scA=scA, scB=scB)

        @pl.when(core == 0)
        def _():
            run(h_a_hbm, eab_hbm)

        @pl.when(core == 1)
        def _():
            run(h_b_hbm, eba_hbm)

        plsc.subcore_barrier()

        def wb(sum_out, cnt_out):
            _per_tile_rows(sub, lambda r0, nr: pltpu.sync_copy(
                acc.at[pl.ds(r0, nr)], sum_out.at[pl.ds(r0, nr)]))

            def cnt_wb(r0, nr):
                pltpu.sync_copy(cnt.at[pl.ds(r0, nr)], stage.at[pl.ds(0, nr)])
                pltpu.sync_copy(stage.at[pl.ds(0, nr)], cnt_out.at[pl.ds(r0, nr)])

            _per_tile_rows(sub, cnt_wb)

        @pl.when(core == 0)
        def _():
            wb(sum_b_out, cnt_b_out)

        @pl.when(core == 1)
        def _():
            wb(sum_a_out, cnt_a_out)

    return k(h_a, h_b, e_ab, e_ba, z128)


def _sc_layer2(h_b1, e_ba, z128):
    mesh = plsc.VectorSubcoreMesh(core_axis_name="c", subcore_axis_name="s")

    @functools.partial(
        pl.kernel,
        out_type=jax.ShapeDtypeStruct((NCORE, N, C), f32),
        mesh=mesh,
        scratch_types=[
            pltpu.VMEM_SHARED((N, C), f32),
            pltpu.VMEM((NPRE, 2, K), i32),
            pltpu.VMEM((K, C), f32),
            pltpu.VMEM((K, C), f32),
            pltpu.SemaphoreType.DMA,
            pltpu.SemaphoreType.DMA,
            pltpu.SemaphoreType.DMA,
            pltpu.SemaphoreType.DMA,
        ],
    )
    def k(h_hbm, eba_hbm, z128_hbm, part_out,
          acc, idx_all, rowsA, rowsB, sgA, sgB, ssA, ssB):
        core = lax.axis_index("c")
        sub = lax.axis_index("s")
        _per_tile_rows(sub, lambda r0, nr: pltpu.sync_copy(
            z128_hbm.at[pl.ds(r0, nr)], acc.at[pl.ds(r0, nr)]))
        plsc.subcore_barrier()

        w = core * NSUB + sub
        c0 = (w * NCHUNK) // (NCORE * NSUB)
        n = ((w + 1) * NCHUNK) // (NCORE * NSUB) - c0
        np_pairs = (NCHUNK // (NCORE * NSUB)) // 2

        _agg_pipeline(h_hbm, eba_hbm, c0, n, np_pairs, NPRE, acc,
                      idx_all, rowsA, rowsB, sgA, sgB, ssA, ssB)

        plsc.subcore_barrier()
        _per_tile_rows(sub, lambda r0, nr: pltpu.sync_copy(
            acc.at[pl.ds(r0, nr)], part_out.at[core, pl.ds(r0, nr)]))

    return k(h_b1, e_ba, z128)


RB = 1000


def _enc_body(xa, xb, wa, ba, wb, bb, ha, hb):
    ha[...] = jnp.maximum(
        jnp.dot(xa[...], wa[...], preferred_element_type=f32) + ba[...], 0.0)
    hb[...] = jnp.maximum(
        jnp.dot(xb[...], wb[...], preferred_element_type=f32) + bb[...], 0.0)


def _encoder(x_a, x_b, W_a, b_a, W_b, b_b):
    row = pl.BlockSpec((RB, C), lambda i: (i, 0))
    w = pl.BlockSpec((C, C), lambda i: (0, 0))
    b = pl.BlockSpec((1, C), lambda i: (0, 0))
    return pl.pallas_call(
        _enc_body,
        grid=(N // RB,),
        in_specs=[row, row, w, b, w, b],
        out_specs=[row, row],
        out_shape=[jax.ShapeDtypeStruct((N, C), f32)] * 2,
    )(x_a, x_b, W_a, b_a, W_b, b_b)


def _l1_body(ha, hb, sa, ca, sb, cb, wsa, wna, wsb, wnb, ha1, hb1):
    ma = sa[...] / jnp.maximum(ca[...], 1.0)
    mb = sb[...] / jnp.maximum(cb[...], 1.0)
    ha1[...] = jnp.maximum(
        jnp.dot(ha[...], wsa[...], preferred_element_type=f32)
        + jnp.dot(ma, wna[...], preferred_element_type=f32), 0.0)
    hb1[...] = jnp.maximum(
        jnp.dot(hb[...], wsb[...], preferred_element_type=f32)
        + jnp.dot(mb, wnb[...], preferred_element_type=f32), 0.0)


def _combine1(h_a, h_b, s_a, c_a, s_b, c_b, wsa, wna, wsb, wnb):
    row = pl.BlockSpec((RB, C), lambda i: (i, 0))
    cnt = pl.BlockSpec((RB, 1), lambda i: (i, 0))
    w = pl.BlockSpec((C, C), lambda i: (0, 0))
    return pl.pallas_call(
        _l1_body,
        grid=(N // RB,),
        in_specs=[row, row, row, cnt, row, cnt, w, w, w, w],
        out_specs=[row, row],
        out_shape=[jax.ShapeDtypeStruct((N, C), f32)] * 2,
    )(h_a, h_b, s_a, c_a, s_b, c_b, wsa, wna, wsb, wnb)


def _l2_body(ha1, p0, p1, ca, ws, wn, wh, bh, out):
    m = (p0[0] + p1[0]) / jnp.maximum(ca[...], 1.0)
    h2 = jnp.maximum(
        jnp.dot(ha1[...], ws[...], preferred_element_type=f32)
        + jnp.dot(m, wn[...], preferred_element_type=f32), 0.0)
    out[...] = jnp.dot(h2, wh[...], preferred_element_type=f32) + bh[...]


def _final(h_a1, parts, c_a, ws, wn, wh, bh):
    row = pl.BlockSpec((RB, C), lambda i: (i, 0))
    p0 = pl.BlockSpec((1, RB, C), lambda i: (0, i, 0))
    p1 = pl.BlockSpec((1, RB, C), lambda i: (1, i, 0))
    cnt = pl.BlockSpec((RB, 1), lambda i: (i, 0))
    w = pl.BlockSpec((C, C), lambda i: (0, 0))
    whs = pl.BlockSpec((C, 1), lambda i: (0, 0))
    bhs = pl.BlockSpec((1, 1), lambda i: (0, 0))
    return pl.pallas_call(
        _l2_body,
        grid=(N // RB,),
        in_specs=[row, p0, p1, cnt, w, w, whs, bhs],
        out_specs=pl.BlockSpec((RB, 1), lambda i: (i, 0)),
        out_shape=jax.ShapeDtypeStruct((N, 1), f32),
    )(h_a1, parts, parts, c_a, ws, wn, wh, bh)


def kernel(x_a, x_b, edge_index_ab, edge_index_ba,
           W_enc_a, b_enc_a, W_enc_b, b_enc_b,
           W_self_a1, W_nbr_a1, W_self_b1, W_nbr_b1,
           W_self_a2, W_nbr_a2, W_self_b2, W_nbr_b2,
           W_head, b_head):
    z128 = jnp.zeros((N, C), f32)
    e_ab = edge_index_ab.reshape(2, NCHUNK, K).transpose(1, 0, 2)
    e_ba = edge_index_ba.reshape(2, NCHUNK, K).transpose(1, 0, 2)

    h_a, h_b = _encoder(x_a, x_b, W_enc_a, b_enc_a.reshape(1, C),
                        W_enc_b, b_enc_b.reshape(1, C))
    sum_b, cnt_b, sum_a, cnt_a = _sc_layer1(h_a, h_b, e_ab, e_ba, z128)
    h_a1, h_b1 = _combine1(h_a, h_b, sum_a, cnt_a[:, None], sum_b,
                           cnt_b[:, None],
                           W_self_a1, W_nbr_a1, W_self_b1, W_nbr_b1)
    parts = _sc_layer2(h_b1, e_ba, z128)
    return _final(h_a1, parts, cnt_a[:, None], W_self_a2, W_nbr_a2,
                  W_head, b_head.reshape(1, 1))
